# baseline (speedup 1.0000x reference)
import math

import jax
import jax.numpy as jnp
import numpy as np
from jax import lax
from jax.experimental import pallas as pl
from jax.experimental.pallas import tpu as pltpu

_N = 224
_HW = _N * _N
_B = 4
_CIN = 192
_CMID = 16
_NBINS = 180
_BW = math.radians(15.0)
_HFR = 0.3
_ALPHA = 1.2
_BETA = 0.8
_PI = math.pi

def _dot1(a, b):
    return jnp.dot(a, b, preferred_element_type=jnp.float32)


def _split(a):
    hi = a.astype(jnp.bfloat16)
    lo = (a - hi.astype(jnp.float32)).astype(jnp.bfloat16)
    return hi, lo


def _dot3(asp, bsp):
    ah, al = asp
    bh, bl = bsp
    return _dot1(ah, bh) + _dot1(ah, bl) + _dot1(al, bh)


def _dot_hi(a, b):
    return _dot3(_split(a), _split(b))


def _dot_lo(a, b):
    return jnp.dot(a.astype(jnp.bfloat16), b.astype(jnp.bfloat16),
                   preferred_element_type=jnp.float32)


def _build_dft():
    N = _N
    j = np.arange(N)
    F = np.exp(-2j * np.pi * np.outer(j, j) / N) / np.sqrt(N)
    Fs = np.roll(F, N // 2, axis=0)
    Fsr = Fs.real.astype(np.float32)
    Fsi = Fs.imag.astype(np.float32)
    return Fsr, Fsi, Fsr.T.copy(), Fsi.T.copy()


(_FSR, _FSI, _FSRT, _FSIT) = _build_dft()


def _build_grids():
    N = _N
    cy, cx = N // 2, N // 2
    y, x = jnp.meshgrid(jnp.arange(N), jnp.arange(N), indexing="ij")
    dy = (y - cy).astype(jnp.float32)
    dx = (x - cx).astype(jnp.float32)
    r = jnp.sqrt(dy ** 2 + dx ** 2)
    theta = jnp.arctan2(dy, dx) + _PI
    r_max = float(min(cy, cx))
    high = (r > _HFR * r_max).astype(jnp.float32)

    theta_m = theta % _PI
    edges = jnp.linspace(0.0, _PI, _NBINS + 1)
    bins = jnp.clip(
        jnp.searchsorted(edges, theta_m.reshape(-1), side="left") - 1,
        0, _NBINS - 1).reshape(N, N).astype(jnp.int32)
    hdiv = high / _CMID
    centers = ((edges[:-1] + edges[1:]) / 2.0).reshape(1, _NBINS)
    return theta, high, hdiv, bins, centers


def _proj_in_kernel(x_ref, w_ref, o_ref):
    o_ref[0] = _dot_hi(w_ref[...], x_ref[0])


def _argmax_rows(e):
    m = jnp.max(e, axis=1, keepdims=True)
    iota = lax.broadcasted_iota(jnp.int32, e.shape, 1)
    return jnp.min(jnp.where(e == m, iota, jnp.int32(2 ** 30)), axis=1,
                   keepdims=True)


def _gather_rows(centers, idx):
    iota = lax.broadcasted_iota(jnp.int32, centers.shape, 1)
    sel = jnp.where(iota == idx, centers, 0.0)
    return jnp.sum(sel, axis=1, keepdims=True)


def _mega_kernel(xp_ref, fsr_ref, fsi_ref, fsrt_ref, fsit_ref,
                 theta_ref, high_ref, hdiv_ref, bins_ref, cent_ref,
                 xh_ref, sr_s, si_s):
    fsr = fsr_ref[...]
    fsi = fsi_ref[...]
    fsrt = fsrt_ref[...]
    fsit = fsit_ref[...]
    fsr_s = _split(fsr)
    fsi_s = _split(fsi)
    fsrt_s = _split(fsrt)
    fsit_s = _split(fsit)

    msum = jnp.zeros((_N, _N), jnp.float32)
    for o in range(_CMID):
        xim_s = _split(xp_ref[0, o])
        tr_s = _split(_dot3(fsr_s, xim_s))
        ti_s = _split(_dot3(fsi_s, xim_s))
        sr = _dot3(tr_s, fsrt_s) - _dot3(ti_s, fsit_s)
        si = _dot3(tr_s, fsit_s) + _dot3(ti_s, fsrt_s)
        sr_s[o] = sr
        si_s[o] = si
        msum = msum + jnp.sqrt(sr * sr + si * si)

    wm = msum * hdiv_ref[...]
    binsv = bins_ref[...]
    i180 = lax.broadcasted_iota(jnp.int32, (1, _NBINS), 1)

    def hist_body(k, acc):
        s = jnp.sum(jnp.where(binsv == k, wm, 0.0))
        return acc + jnp.where(i180 == k, s, 0.0)

    e = lax.fori_loop(0, _NBINS, hist_body,
                      jnp.zeros((1, _NBINS), jnp.float32))

    zero_col = jnp.zeros((1, 1), dtype=e.dtype)
    leftpad = jnp.concatenate([zero_col, e[:, :-1]], axis=1)
    rightpad = jnp.concatenate([e[:, 1:], zero_col], axis=1)
    es = 0.25 * leftpad + 0.5 * e + 0.25 * rightpad
    left = jnp.concatenate([es[:, -1:], es[:, :-1]], axis=1)
    right = jnp.concatenate([es[:, 1:], es[:, :1]], axis=1)
    mean_e = jnp.mean(es, axis=1, keepdims=True)
    mask = (es > mean_e) & (es > left) & (es > right)
    neg_inf = jnp.float32(-jnp.inf)
    score = jnp.where(mask, es, neg_inf)
    idx1 = _argmax_rows(score)
    iota = lax.broadcasted_iota(jnp.int32, score.shape, 1)
    score2 = jnp.where(iota == idx1, neg_inf, score)
    idx2 = _argmax_rows(score2)
    cnt = jnp.sum(mask.astype(jnp.int32), axis=1, keepdims=True)
    idx_fb = _argmax_rows(es)
    centers = cent_ref[...]
    p_fb = _gather_rows(centers, idx_fb)
    p0 = jnp.where(cnt > 0, _gather_rows(centers, idx1), p_fb)
    p1 = jnp.where(cnt > 1, _gather_rows(centers, idx2), p0)

    theta = theta_ref[...]
    hi = high_ref[...] > 0.5
    d0 = jnp.abs(theta - p0)
    d0 = jnp.minimum(d0, _PI - d0)
    d1 = jnp.abs(theta - p1)
    d1 = jnp.minimum(d1, _PI - d1)
    enh = ((d0 <= _BW) | (d1 <= _BW)) & hi
    gain = jnp.where(enh, jnp.float32(_ALPHA),
                     jnp.where(hi, jnp.float32(_BETA), jnp.float32(1.0)))

    for o in range(_CMID):
        er = sr_s[o] * gain
        ei = si_s[o] * gain
        ur = _dot_lo(fsrt, er) + _dot_lo(fsit, ei)
        ui = _dot_lo(fsrt, ei) - _dot_lo(fsit, er)
        xh_ref[0, o] = _dot_lo(ur, fsr) + _dot_lo(ui, fsi)


def _proj_out_kernel(xe_ref, w_ref, x_ref, o_ref):
    o_ref[0] = x_ref[0] + _dot_lo(w_ref[...], xe_ref[0])


_TILE = 6272
_NT = _HW // _TILE


def kernel(x, W_in, W_out):
    B, C, H, W = x.shape
    xf = x.reshape(B, C, _HW)

    fsr = jnp.asarray(_FSR)
    fsi = jnp.asarray(_FSI)
    fsrt = jnp.asarray(_FSRT)
    fsit = jnp.asarray(_FSIT)
    theta, high, hdiv, bins, centers = _build_grids()

    xp = pl.pallas_call(
        _proj_in_kernel,
        grid=(B, _NT),
        in_specs=[
            pl.BlockSpec((1, C, _TILE), lambda b, t: (b, 0, t)),
            pl.BlockSpec((_CMID, C), lambda b, t: (0, 0)),
        ],
        out_specs=pl.BlockSpec((1, _CMID, _TILE), lambda b, t: (b, 0, t)),
        out_shape=jax.ShapeDtypeStruct((B, _CMID, _HW), jnp.float32),
    )(xf, W_in)

    full = pl.BlockSpec((_N, _N), lambda b: (0, 0))
    xh = pl.pallas_call(
        _mega_kernel,
        grid=(B,),
        in_specs=[
            pl.BlockSpec((1, _CMID, _N, _N), lambda b: (b, 0, 0, 0)),
            full, full, full, full, full, full, full,
            pl.BlockSpec((_N, _N), lambda b: (0, 0)),
            pl.BlockSpec((1, _NBINS), lambda b: (0, 0)),
        ],
        out_specs=pl.BlockSpec((1, _CMID, _N, _N), lambda b: (b, 0, 0, 0)),
        out_shape=jax.ShapeDtypeStruct((B, _CMID, _N, _N), jnp.float32),
        scratch_shapes=[
            pltpu.VMEM((_CMID, _N, _N), jnp.float32),
            pltpu.VMEM((_CMID, _N, _N), jnp.float32),
        ],
    )(xp.reshape(B, _CMID, _N, _N), fsr, fsi, fsrt, fsit,
      theta, high, hdiv, bins, centers)

    out = pl.pallas_call(
        _proj_out_kernel,
        grid=(B, _NT),
        in_specs=[
            pl.BlockSpec((1, _CMID, _TILE), lambda b, t: (b, 0, t)),
            pl.BlockSpec((C, _CMID), lambda b, t: (0, 0)),
            pl.BlockSpec((1, C, _TILE), lambda b, t: (b, 0, t)),
        ],
        out_specs=pl.BlockSpec((1, C, _TILE), lambda b, t: (b, 0, t)),
        out_shape=jax.ShapeDtypeStruct((B, C, _HW), jnp.float32),
    )(xh.reshape(B, _CMID, _HW), W_out, xf)

    return out.reshape(B, C, H, W)


# speedup vs baseline: 5.6226x; 5.6226x over previous
import math

import jax
import jax.numpy as jnp
import numpy as np
from jax import lax
from jax.experimental import pallas as pl
from jax.experimental.pallas import tpu as pltpu

_N = 224
_HW = _N * _N
_B = 4
_CIN = 192
_CMID = 16
_NBINS = 180
_BW = math.radians(15.0)
_HFR = 0.3
_ALPHA = 1.2
_BETA = 0.8
_PI = math.pi

def _dot1(a, b):
    return jnp.dot(a, b, preferred_element_type=jnp.float32)


def _split(a):
    hi = a.astype(jnp.bfloat16)
    lo = (a - hi.astype(jnp.float32)).astype(jnp.bfloat16)
    return hi, lo


def _dot3(asp, bsp):
    ah, al = asp
    bh, bl = bsp
    return _dot1(ah, bh) + _dot1(ah, bl) + _dot1(al, bh)


def _dot_hi(a, b):
    return _dot3(_split(a), _split(b))


def _dot_lo(a, b):
    return jnp.dot(a.astype(jnp.bfloat16), b.astype(jnp.bfloat16),
                   preferred_element_type=jnp.float32)


def _build_dft():
    N = _N
    j = np.arange(N)
    F = np.exp(-2j * np.pi * np.outer(j, j) / N) / np.sqrt(N)
    Fs = np.roll(F, N // 2, axis=0)
    Fsr = Fs.real.astype(np.float32)
    Fsi = Fs.imag.astype(np.float32)
    return Fsr, Fsi, Fsr.T.copy(), Fsi.T.copy()


(_FSR, _FSI, _FSRT, _FSIT) = _build_dft()


def _build_grids():
    N = _N
    cy, cx = N // 2, N // 2
    y, x = jnp.meshgrid(jnp.arange(N), jnp.arange(N), indexing="ij")
    dy = (y - cy).astype(jnp.float32)
    dx = (x - cx).astype(jnp.float32)
    r = jnp.sqrt(dy ** 2 + dx ** 2)
    theta = jnp.arctan2(dy, dx) + _PI
    r_max = float(min(cy, cx))
    high = (r > _HFR * r_max).astype(jnp.float32)

    theta_m = theta % _PI
    edges = jnp.linspace(0.0, _PI, _NBINS + 1)
    bins = jnp.clip(
        jnp.searchsorted(edges, theta_m.reshape(-1), side="left") - 1,
        0, _NBINS - 1).reshape(N, N).astype(jnp.int32)
    hdiv = high / _CMID
    centers = ((edges[:-1] + edges[1:]) / 2.0).reshape(1, _NBINS)
    return theta, high, hdiv, bins, centers


_GRIDS = jax.jit(_build_grids)()


def _proj_in_kernel(x_ref, w_ref, o_ref):
    o_ref[0] = _dot_hi(w_ref[...], x_ref[0])


def _argmax_rows(e):
    m = jnp.max(e, axis=1, keepdims=True)
    iota = lax.broadcasted_iota(jnp.int32, e.shape, 1)
    return jnp.min(jnp.where(e == m, iota, jnp.int32(2 ** 30)), axis=1,
                   keepdims=True)


def _gather_rows(centers, idx):
    iota = lax.broadcasted_iota(jnp.int32, centers.shape, 1)
    sel = jnp.where(iota == idx, centers, 0.0)
    return jnp.sum(sel, axis=1, keepdims=True)


def _mega_kernel(xp_ref, fsr_ref, fsi_ref, fsrt_ref, fsit_ref,
                 theta_ref, high_ref, hdiv_ref, bins_ref, cent_ref,
                 xh_ref, sr_s, si_s):
    fsr = fsr_ref[...]
    fsi = fsi_ref[...]
    fsrt = fsrt_ref[...]
    fsit = fsit_ref[...]
    fsr_s = _split(fsr)
    fsi_s = _split(fsi)
    fsrt_s = _split(fsrt)
    fsit_s = _split(fsit)

    msum = jnp.zeros((_N, _N), jnp.float32)
    for o in range(_CMID):
        xim_s = _split(xp_ref[0, o])
        tr_s = _split(_dot3(fsr_s, xim_s))
        ti_s = _split(_dot3(fsi_s, xim_s))
        sr = _dot3(tr_s, fsrt_s) - _dot3(ti_s, fsit_s)
        si = _dot3(tr_s, fsit_s) + _dot3(ti_s, fsrt_s)
        sr_s[o] = sr
        si_s[o] = si
        msum = msum + jnp.sqrt(sr * sr + si * si)

    wm = msum * hdiv_ref[...]
    binsv = bins_ref[...]
    i180 = lax.broadcasted_iota(jnp.int32, (1, _NBINS), 1)

    def hist_body(k, acc):
        s = jnp.sum(jnp.where(binsv == k, wm, 0.0))
        return acc + jnp.where(i180 == k, s, 0.0)

    e = lax.fori_loop(0, _NBINS, hist_body,
                      jnp.zeros((1, _NBINS), jnp.float32))

    zero_col = jnp.zeros((1, 1), dtype=e.dtype)
    leftpad = jnp.concatenate([zero_col, e[:, :-1]], axis=1)
    rightpad = jnp.concatenate([e[:, 1:], zero_col], axis=1)
    es = 0.25 * leftpad + 0.5 * e + 0.25 * rightpad
    left = jnp.concatenate([es[:, -1:], es[:, :-1]], axis=1)
    right = jnp.concatenate([es[:, 1:], es[:, :1]], axis=1)
    mean_e = jnp.mean(es, axis=1, keepdims=True)
    mask = (es > mean_e) & (es > left) & (es > right)
    neg_inf = jnp.float32(-jnp.inf)
    score = jnp.where(mask, es, neg_inf)
    idx1 = _argmax_rows(score)
    iota = lax.broadcasted_iota(jnp.int32, score.shape, 1)
    score2 = jnp.where(iota == idx1, neg_inf, score)
    idx2 = _argmax_rows(score2)
    cnt = jnp.sum(mask.astype(jnp.int32), axis=1, keepdims=True)
    idx_fb = _argmax_rows(es)
    centers = cent_ref[...]
    p_fb = _gather_rows(centers, idx_fb)
    p0 = jnp.where(cnt > 0, _gather_rows(centers, idx1), p_fb)
    p1 = jnp.where(cnt > 1, _gather_rows(centers, idx2), p0)

    theta = theta_ref[...]
    hi = high_ref[...] > 0.5
    d0 = jnp.abs(theta - p0)
    d0 = jnp.minimum(d0, _PI - d0)
    d1 = jnp.abs(theta - p1)
    d1 = jnp.minimum(d1, _PI - d1)
    enh = ((d0 <= _BW) | (d1 <= _BW)) & hi
    gain = jnp.where(enh, jnp.float32(_ALPHA),
                     jnp.where(hi, jnp.float32(_BETA), jnp.float32(1.0)))

    for o in range(_CMID):
        er = sr_s[o] * gain
        ei = si_s[o] * gain
        ur = _dot_lo(fsrt, er) + _dot_lo(fsit, ei)
        ui = _dot_lo(fsrt, ei) - _dot_lo(fsit, er)
        xh_ref[0, o] = _dot_lo(ur, fsr) + _dot_lo(ui, fsi)


def _proj_out_kernel(xe_ref, w_ref, x_ref, o_ref):
    o_ref[0] = x_ref[0] + _dot_lo(w_ref[...], xe_ref[0])


_TILE = 6272
_NT = _HW // _TILE


def kernel(x, W_in, W_out):
    B, C, H, W = x.shape
    xf = x.reshape(B, C, _HW)

    fsr = jnp.asarray(_FSR)
    fsi = jnp.asarray(_FSI)
    fsrt = jnp.asarray(_FSRT)
    fsit = jnp.asarray(_FSIT)
    theta, high, hdiv, bins, centers = _GRIDS

    xp = pl.pallas_call(
        _proj_in_kernel,
        grid=(B, _NT),
        in_specs=[
            pl.BlockSpec((1, C, _TILE), lambda b, t: (b, 0, t)),
            pl.BlockSpec((_CMID, C), lambda b, t: (0, 0)),
        ],
        out_specs=pl.BlockSpec((1, _CMID, _TILE), lambda b, t: (b, 0, t)),
        out_shape=jax.ShapeDtypeStruct((B, _CMID, _HW), jnp.float32),
    )(xf, W_in)

    full = pl.BlockSpec((_N, _N), lambda b: (0, 0))
    xh = pl.pallas_call(
        _mega_kernel,
        grid=(B,),
        in_specs=[
            pl.BlockSpec((1, _CMID, _N, _N), lambda b: (b, 0, 0, 0)),
            full, full, full, full, full, full, full,
            pl.BlockSpec((_N, _N), lambda b: (0, 0)),
            pl.BlockSpec((1, _NBINS), lambda b: (0, 0)),
        ],
        out_specs=pl.BlockSpec((1, _CMID, _N, _N), lambda b: (b, 0, 0, 0)),
        out_shape=jax.ShapeDtypeStruct((B, _CMID, _N, _N), jnp.float32),
        scratch_shapes=[
            pltpu.VMEM((_CMID, _N, _N), jnp.float32),
            pltpu.VMEM((_CMID, _N, _N), jnp.float32),
        ],
    )(xp.reshape(B, _CMID, _N, _N), fsr, fsi, fsrt, fsit,
      theta, high, hdiv, bins, centers)

    out = pl.pallas_call(
        _proj_out_kernel,
        grid=(B, _NT),
        in_specs=[
            pl.BlockSpec((1, _CMID, _TILE), lambda b, t: (b, 0, t)),
            pl.BlockSpec((C, _CMID), lambda b, t: (0, 0)),
            pl.BlockSpec((1, C, _TILE), lambda b, t: (b, 0, t)),
        ],
        out_specs=pl.BlockSpec((1, C, _TILE), lambda b, t: (b, 0, t)),
        out_shape=jax.ShapeDtypeStruct((B, C, _HW), jnp.float32),
    )(xh.reshape(B, _CMID, _HW), W_out, xf)

    return out.reshape(B, C, H, W)


# speedup vs baseline: 6.4584x; 1.1486x over previous
import functools
import math

import jax
import jax.numpy as jnp
import numpy as np
from jax import lax
from jax.experimental import pallas as pl
from jax.experimental.pallas import tpu as pltpu
from jax.experimental.pallas import tpu_sc as plsc

_N = 224
_HW = _N * _N
_B = 4
_CIN = 192
_CMID = 16
_NBINS = 180
_NBINS_PAD = 192
_BW = math.radians(15.0)
_HFR = 0.3
_ALPHA = 1.2
_BETA = 0.8
_PI = math.pi


def _dot1(a, b):
    return jnp.dot(a, b, preferred_element_type=jnp.float32)


def _split(a):
    hi = a.astype(jnp.bfloat16)
    lo = (a - hi.astype(jnp.float32)).astype(jnp.bfloat16)
    return hi, lo


def _dot3(asp, bsp):
    ah, al = asp
    bh, bl = bsp
    return _dot1(ah, bh) + _dot1(ah, bl) + _dot1(al, bh)


def _dot_hi(a, b):
    return _dot3(_split(a), _split(b))


def _dot_lo(a, b):
    return jnp.dot(a.astype(jnp.bfloat16), b.astype(jnp.bfloat16),
                   preferred_element_type=jnp.float32)


def _build_dft():
    N = _N
    j = np.arange(N)
    F = np.exp(-2j * np.pi * np.outer(j, j) / N) / np.sqrt(N)
    Fs = np.roll(F, N // 2, axis=0)
    Fsr = Fs.real.astype(np.float32)
    Fsi = Fs.imag.astype(np.float32)
    return Fsr, Fsi, Fsr.T.copy(), Fsi.T.copy()


(_FSR, _FSI, _FSRT, _FSIT) = _build_dft()


def _build_grids():
    N = _N
    cy, cx = N // 2, N // 2
    y, x = jnp.meshgrid(jnp.arange(N), jnp.arange(N), indexing="ij")
    dy = (y - cy).astype(jnp.float32)
    dx = (x - cx).astype(jnp.float32)
    r = jnp.sqrt(dy ** 2 + dx ** 2)
    theta = jnp.arctan2(dy, dx) + _PI
    r_max = float(min(cy, cx))
    high = (r > _HFR * r_max).astype(jnp.float32)

    theta_m = theta % _PI
    edges = jnp.linspace(0.0, _PI, _NBINS + 1)
    bins = jnp.clip(
        jnp.searchsorted(edges, theta_m.reshape(-1), side="left") - 1,
        0, _NBINS - 1).astype(jnp.int32)
    hdiv = high / _CMID
    centers = ((edges[:-1] + edges[1:]) / 2.0).reshape(1, _NBINS)
    return theta, high, hdiv, bins, centers


_GRIDS_CACHE = []


def _grids_cached():
    if not _GRIDS_CACHE:
        with jax.ensure_compile_time_eval():
            _GRIDS_CACHE.append(_build_grids())
    return _GRIDS_CACHE[0]


def _proj_in_kernel(x_ref, w_ref, o_ref):
    o_ref[0] = _dot_hi(w_ref[...], x_ref[0])


def _argmax_rows(e):
    m = jnp.max(e, axis=1, keepdims=True)
    iota = lax.broadcasted_iota(jnp.int32, e.shape, 1)
    return jnp.min(jnp.where(e == m, iota, jnp.int32(2 ** 30)), axis=1,
                   keepdims=True)


def _gather_rows(centers, idx):
    iota = lax.broadcasted_iota(jnp.int32, centers.shape, 1)
    sel = jnp.where(iota == idx, centers, 0.0)
    return jnp.sum(sel, axis=1, keepdims=True)


def _megaA_kernel(xp_ref, fsr_ref, fsi_ref, fsrt_ref, fsit_ref, hdiv_ref,
                  sr_ref, si_ref, wm_ref):
    fsr_s = _split(fsr_ref[...])
    fsi_s = _split(fsi_ref[...])
    fsrt_s = _split(fsrt_ref[...])
    fsit_s = _split(fsit_ref[...])

    msum = jnp.zeros((_N, _N), jnp.float32)
    for o in range(_CMID):
        xim_s = _split(xp_ref[0, o])
        tr_s = _split(_dot3(fsr_s, xim_s))
        ti_s = _split(_dot3(fsi_s, xim_s))
        sr = _dot3(tr_s, fsrt_s) - _dot3(ti_s, fsit_s)
        si = _dot3(tr_s, fsit_s) + _dot3(ti_s, fsrt_s)
        sr_ref[0, o] = sr
        si_ref[0, o] = si
        msum = msum + jnp.sqrt(sr * sr + si * si)
    wm_ref[0] = msum * hdiv_ref[...]


def _sc_hist(wm4, bins):
    info = plsc.get_sparse_core_info()
    nc, ns = info.num_cores, info.num_subcores
    nw = nc * ns
    chunk = _HW // nw
    mesh = plsc.VectorSubcoreMesh(core_axis_name="c", subcore_axis_name="s")

    seg = _B * _NBINS_PAD

    @functools.partial(
        pl.kernel, mesh=mesh,
        out_type=jax.ShapeDtypeStruct((nc * seg,), jnp.float32),
        scratch_types=[
            pltpu.VMEM((chunk,), jnp.int32),
            pltpu.VMEM((chunk,), jnp.float32),
            pltpu.VMEM((seg,), jnp.float32),
        ] + [pltpu.VMEM_SHARED((_NBINS_PAD,), jnp.float32)
             for _ in range(_B)],
    )
    def k(wm_hbm, bins_hbm, out_hbm, idx_v, wm_v, stage_v, sh0, sh1, sh2, sh3):
        shared = [sh0, sh1, sh2, sh3]
        cid = lax.axis_index("c")
        sid = lax.axis_index("s")
        wid = sid * nc + cid
        base = wid * chunk

        @pl.when(sid == 0)
        def _():
            for i in range(seg // 16):
                stage_v[pl.ds(i * 16, 16)] = jnp.zeros((16,), jnp.float32)
            for b in range(_B):
                pltpu.sync_copy(stage_v.at[pl.ds(b * _NBINS_PAD, _NBINS_PAD)],
                                shared[b])

        plsc.subcore_barrier()
        pltpu.sync_copy(bins_hbm.at[pl.ds(base, chunk)], idx_v)
        for b in range(_B):
            pltpu.sync_copy(wm_hbm.at[pl.ds(b * _HW + base, chunk)], wm_v)
            pltpu.sync_copy(wm_v, shared[b].at[idx_v], add=True)
        plsc.subcore_barrier()

        @pl.when(sid == 0)
        def _():
            for b in range(_B):
                pltpu.sync_copy(shared[b],
                                stage_v.at[pl.ds(b * _NBINS_PAD, _NBINS_PAD)])
            pltpu.sync_copy(stage_v, out_hbm.at[pl.ds(cid * seg, seg)])

    return k(wm4.reshape(_B * _HW), bins).reshape(nc, _B, _NBINS_PAD)


def _peaks_kernel(e_ref, c_ref, p_ref):
    e = (e_ref[0] + e_ref[1])[:, :_NBINS]
    zero_col = jnp.zeros((e.shape[0], 1), dtype=e.dtype)
    leftpad = jnp.concatenate([zero_col, e[:, :-1]], axis=1)
    rightpad = jnp.concatenate([e[:, 1:], zero_col], axis=1)
    es = 0.25 * leftpad + 0.5 * e + 0.25 * rightpad
    left = jnp.concatenate([es[:, -1:], es[:, :-1]], axis=1)
    right = jnp.concatenate([es[:, 1:], es[:, :1]], axis=1)
    mean_e = jnp.mean(es, axis=1, keepdims=True)
    mask = (es > mean_e) & (es > left) & (es > right)
    neg_inf = jnp.float32(-jnp.inf)
    score = jnp.where(mask, es, neg_inf)
    idx1 = _argmax_rows(score)
    iota = lax.broadcasted_iota(jnp.int32, score.shape, 1)
    score2 = jnp.where(iota == idx1, neg_inf, score)
    idx2 = _argmax_rows(score2)
    cnt = jnp.sum(mask.astype(jnp.int32), axis=1, keepdims=True)
    idx_fb = _argmax_rows(es)
    centers = jnp.broadcast_to(c_ref[...], (e.shape[0], _NBINS))
    p_fb = _gather_rows(centers, idx_fb)
    p0 = jnp.where(cnt > 0, _gather_rows(centers, idx1), p_fb)
    p1 = jnp.where(cnt > 1, _gather_rows(centers, idx2), p0)
    p_ref[...] = jnp.concatenate([p0, p1], axis=1)


def _megaB_kernel(sr_ref, si_ref, pk_ref, th_ref, hi_ref,
                  fsr_ref, fsi_ref, fsrt_ref, fsit_ref, xh_ref):
    b = pl.program_id(0)
    theta = th_ref[...]
    hi = hi_ref[...] > 0.5
    p0 = pk_ref[b, 0]
    p1 = pk_ref[b, 1]
    d0 = jnp.abs(theta - p0)
    d0 = jnp.minimum(d0, _PI - d0)
    d1 = jnp.abs(theta - p1)
    d1 = jnp.minimum(d1, _PI - d1)
    enh = ((d0 <= _BW) | (d1 <= _BW)) & hi
    gain = jnp.where(enh, jnp.float32(_ALPHA),
                     jnp.where(hi, jnp.float32(_BETA), jnp.float32(1.0)))

    fsr = fsr_ref[...]
    fsi = fsi_ref[...]
    fsrt = fsrt_ref[...]
    fsit = fsit_ref[...]
    for o in range(_CMID):
        er = sr_ref[0, o] * gain
        ei = si_ref[0, o] * gain
        ur = _dot_lo(fsrt, er) + _dot_lo(fsit, ei)
        ui = _dot_lo(fsrt, ei) - _dot_lo(fsit, er)
        xh_ref[0, o] = _dot_lo(ur, fsr) + _dot_lo(ui, fsi)


def _proj_out_kernel(xe_ref, w_ref, x_ref, o_ref):
    o_ref[0] = x_ref[0] + _dot_lo(w_ref[...], xe_ref[0])


_TILE = 6272
_NT = _HW // _TILE


def kernel(x, W_in, W_out):
    B, C, H, W = x.shape
    xf = x.reshape(B, C, _HW)

    fsr = jnp.asarray(_FSR)
    fsi = jnp.asarray(_FSI)
    fsrt = jnp.asarray(_FSRT)
    fsit = jnp.asarray(_FSIT)
    theta, high, hdiv, bins, centers = _grids_cached()

    xp = pl.pallas_call(
        _proj_in_kernel,
        grid=(B, _NT),
        in_specs=[
            pl.BlockSpec((1, C, _TILE), lambda b, t: (b, 0, t)),
            pl.BlockSpec((_CMID, C), lambda b, t: (0, 0)),
        ],
        out_specs=pl.BlockSpec((1, _CMID, _TILE), lambda b, t: (b, 0, t)),
        out_shape=jax.ShapeDtypeStruct((B, _CMID, _HW), jnp.float32),
    )(xf, W_in)

    full = pl.BlockSpec((_N, _N), lambda b: (0, 0))
    img = pl.BlockSpec((1, _CMID, _N, _N), lambda b: (b, 0, 0, 0))
    sr, si, wm = pl.pallas_call(
        _megaA_kernel,
        grid=(B,),
        in_specs=[img, full, full, full, full, full],
        out_specs=[img, img, pl.BlockSpec((1, _N, _N), lambda b: (b, 0, 0))],
        out_shape=[
            jax.ShapeDtypeStruct((B, _CMID, _N, _N), jnp.float32),
            jax.ShapeDtypeStruct((B, _CMID, _N, _N), jnp.float32),
            jax.ShapeDtypeStruct((B, _N, _N), jnp.float32),
        ],
    )(xp.reshape(B, _CMID, _N, _N), fsr, fsi, fsrt, fsit, hdiv)

    epart = _sc_hist(wm.reshape(B, _HW), bins)

    peaks = pl.pallas_call(
        _peaks_kernel,
        in_specs=[
            pl.BlockSpec((2, B, _NBINS_PAD), lambda: (0, 0, 0)),
            pl.BlockSpec((1, _NBINS), lambda: (0, 0)),
        ],
        out_specs=pl.BlockSpec((B, 2), lambda: (0, 0)),
        out_shape=jax.ShapeDtypeStruct((B, 2), jnp.float32),
    )(epart, centers)

    xh = pl.pallas_call(
        _megaB_kernel,
        grid=(B,),
        in_specs=[img, img,
                  pl.BlockSpec((B, 2), lambda b: (0, 0)),
                  full, full, full, full, full, full],
        out_specs=img,
        out_shape=jax.ShapeDtypeStruct((B, _CMID, _N, _N), jnp.float32),
    )(sr, si, peaks, theta, high, fsr, fsi, fsrt, fsit)

    out = pl.pallas_call(
        _proj_out_kernel,
        grid=(B, _NT),
        in_specs=[
            pl.BlockSpec((1, _CMID, _TILE), lambda b, t: (b, 0, t)),
            pl.BlockSpec((C, _CMID), lambda b, t: (0, 0)),
            pl.BlockSpec((1, C, _TILE), lambda b, t: (b, 0, t)),
        ],
        out_specs=pl.BlockSpec((1, C, _TILE), lambda b, t: (b, 0, t)),
        out_shape=jax.ShapeDtypeStruct((B, C, _HW), jnp.float32),
    )(xh.reshape(B, _CMID, _HW), W_out, xf)

    return out.reshape(B, C, H, W)
